# baseline (speedup 1.0000x reference)
import jax
import jax.numpy as jnp
from jax import lax
from jax.experimental import pallas as pl
from jax.experimental.pallas import tpu as pltpu
from jax.experimental.pallas import tpu_sc as plsc

N = 10000
D = 128
NUM_GRAPHS = 64
NUM_ELEMENTS = 100
NUM_KEYS = NUM_GRAPHS * NUM_ELEMENTS
HALF_KEYS = NUM_KEYS // 2
NUM_TILES = 16
CHUNK = 128
ACC_ROWS = 3328
ZERO_ROWS = ACC_ROWS // NUM_TILES
OUT_ROWS_PER_TILE = HALF_KEYS // NUM_TILES


def _chunked_copy(src_at, dst_at, total):
    done = 0
    while done < total:
        step = min(CHUNK, total - done)
        pltpu.sync_copy(src_at(done, step), dst_at(done, step))
        done += step


def _sc_body(out_hbm, z_hbm, b_hbm, split_hbm, o_hbm,
             rowbuf, zbuf, bbuf, keybuf, splitbuf, acc):
    c = lax.axis_index("c")
    s = lax.axis_index("s")

    def _zero_row(i, carry):
        for j in range(D // 16):
            rowbuf[i, pl.ds(j * 16, 16)] = jnp.zeros((16,), jnp.float32)
        return carry

    lax.fori_loop(0, CHUNK, _zero_row, 0)
    zbase = s * ZERO_ROWS
    _chunked_copy(lambda off, n: rowbuf.at[pl.ds(0, n)],
                  lambda off, n: acc.at[pl.ds(zbase + off, n)], ZERO_ROWS)

    pltpu.sync_copy(split_hbm, splitbuf)
    split = splitbuf[...][0]
    split8 = (split // 8) * 8
    is0 = c == 0
    cnt = jnp.where(is0, split, N - split8)
    base = jnp.where(is0, 0, split8)
    row_lo = jnp.where(is0, 0, split)
    row_hi = jnp.where(is0, split, N)
    m = (cnt + CHUNK - 1) // CHUNK
    rpt = m * 8
    nch = (m + NUM_TILES - 1) // NUM_TILES

    lo = base + s * rpt
    tile_hi = jnp.minimum(lo + rpt, row_hi)
    trash = HALF_KEYS + s
    key_base = c * HALF_KEYS

    plsc.subcore_barrier()

    def _chunk(k, carry):
        start_l = lo + k * CHUNK
        a = jnp.minimum(start_l, N - CHUNK)
        win_lo = jnp.maximum(start_l, row_lo)
        win_hi = jnp.minimum(start_l + CHUNK, tile_hi)
        pltpu.sync_copy(out_hbm.at[pl.ds(a, CHUNK)], rowbuf)
        pltpu.sync_copy(z_hbm.at[pl.ds(a, CHUNK)], zbuf)
        pltpu.sync_copy(b_hbm.at[pl.ds(a, CHUNK)], bbuf)
        for j in range(CHUNK // 16):
            zv = zbuf[pl.ds(j * 16, 16)]
            bv = bbuf[pl.ds(j * 16, 16)]
            grow = a + j * 16 + lax.iota(jnp.int32, 16)
            key = bv * NUM_ELEMENTS + zv - 1 - key_base
            valid = ((grow >= win_lo) & (grow < win_hi)
                     & (key >= 0) & (key < HALF_KEYS))
            keybuf[pl.ds(j * 16, 16)] = jnp.where(valid, key, trash)
        pltpu.sync_copy(rowbuf, acc.at[keybuf], add=True)
        return carry

    lax.fori_loop(0, nch, _chunk, 0)
    plsc.subcore_barrier()

    obase = s * OUT_ROWS_PER_TILE
    _chunked_copy(lambda off, n: acc.at[pl.ds(obase + off, n)],
                  lambda off, n: o_hbm.at[pl.ds(c * HALF_KEYS + obase + off, n)],
                  OUT_ROWS_PER_TILE)


@jax.jit
def _pool_call(out, z32, b32):
    split = jnp.searchsorted(b32, jnp.int32(NUM_GRAPHS // 2)).astype(jnp.int32)
    splits = jnp.full((16,), split, jnp.int32)
    mesh = plsc.VectorSubcoreMesh(core_axis_name="c", subcore_axis_name="s")
    return pl.kernel(
        _sc_body,
        out_type=jax.ShapeDtypeStruct((NUM_KEYS, D), jnp.float32),
        mesh=mesh,
        scratch_types=[
            pltpu.VMEM((CHUNK, D), jnp.float32),
            pltpu.VMEM((CHUNK,), jnp.int32),
            pltpu.VMEM((CHUNK,), jnp.int32),
            pltpu.VMEM((CHUNK,), jnp.int32),
            pltpu.VMEM((16,), jnp.int32),
            pltpu.VMEM_SHARED((ACC_ROWS, D), jnp.float32),
        ],
    )(out, z32, b32, splits)


def kernel(out, z, batch):
    pooled = _pool_call(out, z.astype(jnp.int32), batch.astype(jnp.int32))
    return pooled.reshape(NUM_GRAPHS, NUM_ELEMENTS * D)


# speedup vs baseline: 1.6486x; 1.6486x over previous
import jax
import jax.numpy as jnp
from jax import lax
from jax.experimental import pallas as pl
from jax.experimental.pallas import tpu as pltpu
from jax.experimental.pallas import tpu_sc as plsc

N = 10000
D = 128
NUM_GRAPHS = 64
NUM_ELEMENTS = 100
NUM_KEYS = NUM_GRAPHS * NUM_ELEMENTS
HALF_KEYS = NUM_KEYS // 2
NUM_TILES = 16
CHUNK = 128
ACC_ROWS = 3328
ZERO_ROWS = ACC_ROWS // NUM_TILES
OUT_ROWS_PER_TILE = HALF_KEYS // NUM_TILES


def _chunked_copy(src_at, dst_at, total):
    done = 0
    while done < total:
        step = min(CHUNK, total - done)
        pltpu.sync_copy(src_at(done, step), dst_at(done, step))
        done += step


def _sc_body(out_hbm, z_hbm, b_hbm, split_hbm, o_hbm,
             rowbuf, zbuf, bbuf, keybuf, splitbuf, acc):
    c = lax.axis_index("c")
    s = lax.axis_index("s")

    def _zero_row(i, carry):
        for j in range(D // 16):
            rowbuf[i, pl.ds(j * 16, 16)] = jnp.zeros((16,), jnp.float32)
        return carry

    lax.fori_loop(0, CHUNK, _zero_row, 0)
    zbase = s * ZERO_ROWS
    _chunked_copy(lambda off, n: rowbuf.at[pl.ds(0, n)],
                  lambda off, n: acc.at[pl.ds(zbase + off, n)], ZERO_ROWS)

    pltpu.sync_copy(split_hbm, splitbuf)
    split = splitbuf[...][0]
    split8 = (split // 8) * 8
    is0 = c == 0
    cnt = jnp.where(is0, split, N - split8)
    base = jnp.where(is0, 0, split8)
    row_lo = jnp.where(is0, 0, split)
    row_hi = jnp.where(is0, split, N)
    m = (cnt + CHUNK - 1) // CHUNK
    rpt = m * 8
    nch = (m + NUM_TILES - 1) // NUM_TILES

    lo = base + s * rpt
    tile_hi = jnp.minimum(lo + rpt, row_hi)
    trash = HALF_KEYS + s
    key_base = c * HALF_KEYS

    plsc.subcore_barrier()

    def _chunk(k, carry):
        start_l = lo + k * CHUNK
        a = jnp.minimum(start_l, N - CHUNK)
        win_lo = jnp.maximum(start_l, row_lo)
        win_hi = jnp.minimum(start_l + CHUNK, tile_hi)
        pltpu.sync_copy(out_hbm.at[pl.ds(a, CHUNK)], rowbuf)
        pltpu.sync_copy(z_hbm.at[pl.ds(a, CHUNK)], zbuf)
        pltpu.sync_copy(b_hbm.at[pl.ds(a, CHUNK)], bbuf)
        for j in range(CHUNK // 16):
            zv = zbuf[pl.ds(j * 16, 16)]
            bv = bbuf[pl.ds(j * 16, 16)]
            grow = a + j * 16 + lax.iota(jnp.int32, 16)
            key = bv * NUM_ELEMENTS + zv - 1 - key_base
            valid = ((grow >= win_lo) & (grow < win_hi)
                     & (key >= 0) & (key < HALF_KEYS))
            keybuf[pl.ds(j * 16, 16)] = jnp.where(valid, key, trash)
        pltpu.sync_copy(rowbuf, acc.at[keybuf], add=True)
        return carry

    lax.fori_loop(0, nch, _chunk, 0)
    plsc.subcore_barrier()

    obase = s * OUT_ROWS_PER_TILE
    _chunked_copy(lambda off, n: acc.at[pl.ds(obase + off, n)],
                  lambda off, n: o_hbm.at[pl.ds(c * HALF_KEYS + obase + off, n)],
                  OUT_ROWS_PER_TILE)


@jax.jit
def _pool_call(out, z32, b32):
    split = jnp.sum((b32 < NUM_GRAPHS // 2).astype(jnp.int32)).astype(jnp.int32)
    splits = jnp.full((16,), split, jnp.int32)
    mesh = plsc.VectorSubcoreMesh(core_axis_name="c", subcore_axis_name="s")
    return pl.kernel(
        _sc_body,
        out_type=jax.ShapeDtypeStruct((NUM_KEYS, D), jnp.float32),
        mesh=mesh,
        scratch_types=[
            pltpu.VMEM((CHUNK, D), jnp.float32),
            pltpu.VMEM((CHUNK,), jnp.int32),
            pltpu.VMEM((CHUNK,), jnp.int32),
            pltpu.VMEM((CHUNK,), jnp.int32),
            pltpu.VMEM((16,), jnp.int32),
            pltpu.VMEM_SHARED((ACC_ROWS, D), jnp.float32),
        ],
    )(out, z32, b32, splits)


def kernel(out, z, batch):
    pooled = _pool_call(out, z.astype(jnp.int32), batch.astype(jnp.int32))
    return pooled.reshape(NUM_GRAPHS, NUM_ELEMENTS * D)


# speedup vs baseline: 1.6511x; 1.0015x over previous
import jax
import jax.numpy as jnp
from jax import lax
from jax.experimental import pallas as pl
from jax.experimental.pallas import tpu as pltpu
from jax.experimental.pallas import tpu_sc as plsc

N = 10000
D = 128
NUM_GRAPHS = 64
NUM_ELEMENTS = 100
NUM_KEYS = NUM_GRAPHS * NUM_ELEMENTS
HALF_KEYS = NUM_KEYS // 2
NUM_TILES = 16
CHUNK = 128
ACC_ROWS = 3328
ZERO_ROWS = ACC_ROWS // NUM_TILES
OUT_ROWS_PER_TILE = HALF_KEYS // NUM_TILES


def _chunked_copy(src_at, dst_at, total):
    done = 0
    while done < total:
        step = min(CHUNK, total - done)
        pltpu.sync_copy(src_at(done, step), dst_at(done, step))
        done += step


def _sc_body(out_hbm, z_hbm, b_hbm, split_hbm, zeros_hbm, o_hbm,
             rb0, zb0, bb0, rb1, zb1, bb1, keybuf, splitbuf, acc,
             sem0, sem1, semz):
    c = lax.axis_index("c")
    s = lax.axis_index("s")

    pltpu.sync_copy(split_hbm, splitbuf)
    split = splitbuf[...][0]
    split8 = (split // 8) * 8
    is0 = c == 0
    cnt = jnp.where(is0, split, N - split8)
    base = jnp.where(is0, 0, split8)
    row_lo = jnp.where(is0, 0, split)
    row_hi = jnp.where(is0, split, N)
    m = (cnt + CHUNK - 1) // CHUNK
    cs = (m * s) // NUM_TILES
    ce = (m * (s + 1)) // NUM_TILES
    nch = ce - cs
    trash = HALF_KEYS + s
    key_base = c * HALF_KEYS

    def _load_addr(k):
        return jnp.minimum(base + (cs + k) * CHUNK, N - CHUNK)

    def _start(k, rb, zb, bb, sem):
        a = _load_addr(k)
        pltpu.async_copy(out_hbm.at[pl.ds(a, CHUNK)], rb, sem)
        pltpu.async_copy(z_hbm.at[pl.ds(a, CHUNK)], zb, sem)
        pltpu.async_copy(b_hbm.at[pl.ds(a, CHUNK)], bb, sem)

    def _wait(rb, zb, bb, sem):
        pltpu.make_async_copy(out_hbm.at[pl.ds(0, CHUNK)], rb, sem).wait()
        pltpu.make_async_copy(z_hbm.at[pl.ds(0, CHUNK)], zb, sem).wait()
        pltpu.make_async_copy(b_hbm.at[pl.ds(0, CHUNK)], bb, sem).wait()

    def _process(k, rb, zb, bb):
        start_l = base + (cs + k) * CHUNK
        a = _load_addr(k)
        win_lo = jnp.maximum(start_l, row_lo)
        win_hi = jnp.minimum(start_l + CHUNK, row_hi)
        for j in range(CHUNK // 16):
            zv = zb[pl.ds(j * 16, 16)]
            bv = bb[pl.ds(j * 16, 16)]
            grow = a + j * 16 + lax.iota(jnp.int32, 16)
            key = bv * NUM_ELEMENTS + zv - 1 - key_base
            valid = ((grow >= win_lo) & (grow < win_hi)
                     & (key >= 0) & (key < HALF_KEYS))
            keybuf[pl.ds(j * 16, 16)] = jnp.where(valid, key, trash)
        pltpu.sync_copy(rb, acc.at[keybuf], add=True)

    _start(0, rb0, zb0, bb0, sem0)
    zd = pltpu.async_copy(zeros_hbm, acc.at[pl.ds(s * ZERO_ROWS, ZERO_ROWS)],
                          semz)
    zd.wait()
    plsc.subcore_barrier()

    def _pair(i, carry):
        k0 = 2 * i
        _wait(rb0, zb0, bb0, sem0)
        _start(k0 + 1, rb1, zb1, bb1, sem1)

        @pl.when(k0 < nch)
        def _():
            _process(k0, rb0, zb0, bb0)

        _wait(rb1, zb1, bb1, sem1)
        _start(k0 + 2, rb0, zb0, bb0, sem0)

        @pl.when(k0 + 1 < nch)
        def _():
            _process(k0 + 1, rb1, zb1, bb1)

        return carry

    lax.fori_loop(0, (nch + 1) // 2, _pair, 0)
    _wait(rb0, zb0, bb0, sem0)
    plsc.subcore_barrier()

    obase = s * OUT_ROWS_PER_TILE
    _chunked_copy(lambda off, n: acc.at[pl.ds(obase + off, n)],
                  lambda off, n: o_hbm.at[pl.ds(c * HALF_KEYS + obase + off, n)],
                  OUT_ROWS_PER_TILE)


@jax.jit
def _pool_call(out, z32, b32):
    split = jnp.sum((b32 < NUM_GRAPHS // 2).astype(jnp.int32)).astype(jnp.int32)
    splits = jnp.full((16,), split, jnp.int32)
    zeros = jnp.zeros((ZERO_ROWS, D), jnp.float32)
    mesh = plsc.VectorSubcoreMesh(core_axis_name="c", subcore_axis_name="s")
    return pl.kernel(
        _sc_body,
        out_type=jax.ShapeDtypeStruct((NUM_KEYS, D), jnp.float32),
        mesh=mesh,
        scratch_types=[
            pltpu.VMEM((CHUNK, D), jnp.float32),
            pltpu.VMEM((CHUNK,), jnp.int32),
            pltpu.VMEM((CHUNK,), jnp.int32),
            pltpu.VMEM((CHUNK, D), jnp.float32),
            pltpu.VMEM((CHUNK,), jnp.int32),
            pltpu.VMEM((CHUNK,), jnp.int32),
            pltpu.VMEM((CHUNK,), jnp.int32),
            pltpu.VMEM((16,), jnp.int32),
            pltpu.VMEM_SHARED((ACC_ROWS, D), jnp.float32),
            pltpu.SemaphoreType.DMA,
            pltpu.SemaphoreType.DMA,
            pltpu.SemaphoreType.DMA,
        ],
    )(out, z32, b32, splits, zeros)


def kernel(out, z, batch):
    pooled = _pool_call(out, z.astype(jnp.int32), batch.astype(jnp.int32))
    return pooled.reshape(NUM_GRAPHS, NUM_ELEMENTS * D)
